# Initial kernel scaffold; baseline (speedup 1.0000x reference)
import jax
import jax.numpy as jnp
from jax.experimental import pallas as pl


def kernel(pred_coord_0, pred_coord_1, pred_coord_2, pred_coord_before_deform_0, pred_coord_before_deform_1, pred_coord_before_deform_2, pred_depth, gt_points, gt_normals, gt_images, gt_depth, mask, edges_0, edges_1, edges_2, laplace_idx_0, laplace_idx_1, laplace_idx_2):
    raise NotImplementedError("write your pallas kernel here")


# speedup vs baseline: 5.6166x; 5.6166x over previous
import functools

import jax
import jax.numpy as jnp
from jax import lax
from jax.experimental import pallas as pl
from jax.experimental.pallas import tpu as pltpu
from jax.experimental.pallas import tpu_sc as plsc

_W_CHAMFER = (1.0, 1.0, 1.0)
_W_CHAMFER_OPP = 0.55
_W_LAPLACE = 0.5
_W_MOVE = 0.1
_W_EDGE = 0.1
_W_NORMAL = 0.00016
_W_DEPTH = 1.0
_LAP_CONST = (0.2, 1.0, 1.0)

_B = 2
_NGT = 8000
_NGTP = 8192
_TN = 512
_NT = _NGTP // _TN
_NV = (156, 618, 2466)
_NE = (462, 1848, 7392)
_NPAD = (256, 768, 2560)
_VOFF = (0, 256, 1024)
_MTOT = 3584
_EPAD = (512, 2048, 7424)
_EOFF = (0, 512, 2560)
_ETOT = 9984
_EW = tuple(e // 16 for e in _EPAD)
_ESEG = (0, 32, 160)
_EVW = 624
_NW = tuple(p // 16 for p in _NPAD)
_LSEG = (0, 16, 64)
_LVW = 224
_PADV = 1e6
_NTERMS = 11


def _chamfer_body(gt_ref, pred_ref, pa_ref, idx2_ref, min2_s):
    t = pl.program_id(1)
    g = gt_ref[0]
    p = pred_ref[0]
    cross = lax.dot_general(g, p, (((1,), (0,)), ((), ())),
                            preferred_element_type=jnp.float32)
    g2 = jnp.sum(g * g, axis=1, keepdims=True)
    p2 = jnp.sum(p * p, axis=0, keepdims=True)
    d = (g2 - 2.0 * cross) + p2

    rows = lax.broadcasted_iota(jnp.int32, (_TN, 1), 0) + t * _TN
    rvalid = rows < _NGT
    s1 = []
    for i in range(3):
        m1 = jnp.min(d[:, _VOFF[i]:_VOFF[i] + _NPAD[i]], axis=1, keepdims=True)
        s1.append(jnp.sum(jnp.where(rvalid, m1, 0.0)))

    tile_min = jnp.min(d, axis=0, keepdims=True)
    riota = lax.broadcasted_iota(jnp.int32, (_TN, _MTOT), 0)
    targ = jnp.min(jnp.where(d == tile_min, riota, _TN), axis=0,
                   keepdims=True) + t * _TN

    @pl.when(t == 0)
    def _():
        min2_s[...] = jnp.full((1, _MTOT), 1e30, jnp.float32)

    cur = min2_s[...]
    better = tile_min < cur
    min2_s[...] = jnp.where(better, tile_min, cur)
    idx2_ref[...] = jnp.where(better, targ,
                              idx2_ref[...].reshape(1, _MTOT)
                              ).reshape(1, 1, _MTOT)

    lane = lax.broadcasted_iota(jnp.int32, (1, 128), 1)
    vals = jnp.zeros((1, 128), jnp.float32)
    for i in range(3):
        vals = jnp.where(lane == i, s1[i], vals)

    @pl.when(t == 0)
    def _():
        pa_ref[...] = jnp.zeros((1, 8, 128), jnp.float32)

    @pl.when(t == _NT - 1)
    def _():
        m2 = jnp.where(better, tile_min, cur)
        v2 = jnp.zeros((1, 128), jnp.float32)
        for i in range(3):
            sl = m2[:, _VOFF[i]:_VOFF[i] + _NPAD[i]]
            li = lax.broadcasted_iota(jnp.int32, (1, _NPAD[i]), 1)
            s2 = jnp.sum(jnp.where(li < _NV[i], sl, 0.0))
            v2 = jnp.where(lane == 3 + i, s2, v2)
        subl2 = lax.broadcasted_iota(jnp.int32, (1, 8, 128), 1)
        pa_ref[...] = pa_ref[...] + jnp.where(subl2 == 1,
                                              v2.reshape(1, 1, 128), 0.0)

    subl = lax.broadcasted_iota(jnp.int32, (1, 8, 128), 1)
    v8 = jnp.where(subl == 0, vals.reshape(1, 1, 128), 0.0)
    pa_ref[...] = pa_ref[...] + v8


def _rsqrt16(x):
    i = lax.bitcast_convert_type(x, jnp.int32)
    i = jnp.int32(0x5F3759DF) - lax.shift_right_arithmetic(i, 1)
    y = lax.bitcast_convert_type(i, jnp.float32)
    for _ in range(3):
        y = y * (1.5 - 0.5 * x * y * y)
    return y


def _reg_body(predT, befT, gtnT, idx2, e0_hbm, e1_hbm, lap_hbm, out,
              px, py, pz, bx, by, bz, gnx, gny, gnz, i2v, e0_v, e1_v, lap_v,
              a0, a1, a2, a3, a4, a5, a6, a7, a8, a9, a10, sem):
    c = lax.axis_index("c")
    s = lax.axis_index("s")
    wid = s * 2 + c
    b = wid // 16
    w16 = wid % 16

    copies = []

    def cp(src, dst):
        copies.append(pltpu.async_copy(src, dst, sem))

    cp(predT.at[pl.ds((b * 8 + 0) * _MTOT, _MTOT)], px)
    cp(predT.at[pl.ds((b * 8 + 1) * _MTOT, _MTOT)], py)
    cp(predT.at[pl.ds((b * 8 + 2) * _MTOT, _MTOT)], pz)
    cp(befT.at[pl.ds((b * 8 + 0) * _MTOT, _MTOT)], bx)
    cp(befT.at[pl.ds((b * 8 + 1) * _MTOT, _MTOT)], by)
    cp(befT.at[pl.ds((b * 8 + 2) * _MTOT, _MTOT)], bz)
    cp(gtnT.at[pl.ds((b * 3 + 0) * _NGTP, _NGTP)], gnx)
    cp(gtnT.at[pl.ds((b * 3 + 1) * _NGTP, _NGTP)], gny)
    cp(gtnT.at[pl.ds((b * 3 + 2) * _NGTP, _NGTP)], gnz)
    cp(idx2.at[pl.ds(b * _MTOT, _MTOT)], i2v)
    for i in range(3):
        cp(e0_hbm.at[pl.ds(_EOFF[i] + w16 * _EW[i], _EW[i])],
           e0_v.at[pl.ds(_ESEG[i], _EW[i])])
        cp(e1_hbm.at[pl.ds(_EOFF[i] + w16 * _EW[i], _EW[i])],
           e1_v.at[pl.ds(_ESEG[i], _EW[i])])
        for k in range(8):
            cp(lap_hbm.at[pl.ds(k * _MTOT + _VOFF[i] + w16 * _NW[i], _NW[i])],
               lap_v.at[pl.ds(k * _LVW + _LSEG[i], _NW[i])])
    for cpy in copies:
        cpy.wait()

    zero = jnp.zeros((16,), jnp.float32)
    lane16 = lax.iota(jnp.int32, 16)
    acc_e = [zero, zero, zero]
    acc_n = [zero, zero, zero]
    acc_l = [zero, zero, zero]
    acc_m = [zero, zero, zero]

    for i in range(3):
        for j in range(_EW[i] // 16):
            gid = w16 * _EW[i] + j * 16 + lane16
            valid = gid < _NE[i]
            i0 = e0_v[pl.ds(_ESEG[i] + j * 16, 16)] + _VOFF[i]
            i1 = e1_v[pl.ds(_ESEG[i] + j * 16, 16)] + _VOFF[i]
            p0x = plsc.load_gather(px, [i0])
            p0y = plsc.load_gather(py, [i0])
            p0z = plsc.load_gather(pz, [i0])
            p1x = plsc.load_gather(px, [i1])
            p1y = plsc.load_gather(py, [i1])
            p1z = plsc.load_gather(pz, [i1])
            dx = p0x - p1x
            dy = p0y - p1y
            dz = p0z - p1z
            ss = dx * dx + dy * dy + dz * dz
            acc_e[i] = acc_e[i] + jnp.where(valid, ss, 0.0)
            nrm = ss * _rsqrt16(jnp.maximum(ss, 1e-30))
            inv = 1.0 / jnp.maximum(nrm, 1e-12)
            gi = plsc.load_gather(i2v, [i0])
            nxv = plsc.load_gather(gnx, [gi])
            nyv = plsc.load_gather(gny, [gi])
            nzv = plsc.load_gather(gnz, [gi])
            ssn = nxv * nxv + nyv * nyv + nzv * nzv
            nrmn = ssn * _rsqrt16(jnp.maximum(ssn, 1e-30))
            invn = 1.0 / jnp.maximum(nrmn, 1e-12)
            dot = (dx * nxv + dy * nyv + dz * nzv) * (inv * invn)
            acc_n[i] = acc_n[i] + jnp.where(valid, jnp.abs(dot), 0.0)

    for i in range(3):
        for j in range(_NW[i] // 16):
            vbase = w16 * _NW[i] + j * 16
            vid = vbase + lane16
            valid = vid < _NV[i]
            goff = _VOFF[i] + vbase
            dx = bx[pl.ds(goff, 16)] - px[pl.ds(goff, 16)]
            dy = by[pl.ds(goff, 16)] - py[pl.ds(goff, 16)]
            dz = bz[pl.ds(goff, 16)] - pz[pl.ds(goff, 16)]
            nsx = zero
            nsy = zero
            nsz = zero
            cnt = zero
            for k in range(8):
                nb = lap_v[pl.ds(k * _LVW + _LSEG[i] + j * 16, 16)]
                nvalid = nb >= 0
                safe = jnp.maximum(nb, 0) + _VOFF[i]
                gx = plsc.load_gather(bx, [safe]) - plsc.load_gather(px, [safe])
                gy = plsc.load_gather(by, [safe]) - plsc.load_gather(py, [safe])
                gz = plsc.load_gather(bz, [safe]) - plsc.load_gather(pz, [safe])
                nsx = nsx + jnp.where(nvalid, gx, 0.0)
                nsy = nsy + jnp.where(nvalid, gy, 0.0)
                nsz = nsz + jnp.where(nvalid, gz, 0.0)
                cnt = cnt + jnp.where(nvalid, 1.0, 0.0)
            cc = jnp.maximum(cnt, 1.0)
            ldx = dx - nsx / cc
            ldy = dy - nsy / cc
            ldz = dz - nsz / cc
            lsq = ldx * ldx + ldy * ldy + ldz * ldz
            acc_l[i] = acc_l[i] + jnp.where(valid, lsq, 0.0)
            if i > 0:
                mv = dx * dx + dy * dy + dz * dz
                acc_m[i] = acc_m[i] + jnp.where(valid, mv, 0.0)

    regs = [acc_e[0], acc_e[1], acc_e[2],
            acc_n[0], acc_n[1], acc_n[2],
            acc_l[0], acc_l[1], acc_l[2],
            acc_m[1], acc_m[2]]
    stores = [a0, a1, a2, a3, a4, a5, a6, a7, a8, a9, a10]
    out_copies = []
    for r in range(_NTERMS):
        stores[r][...] = regs[r]
        out_copies.append(pltpu.async_copy(
            stores[r], out.at[pl.ds((wid * _NTERMS + r) * 16, 16)], sem))
    for cpy in out_copies:
        cpy.wait()


def _final_body(pa_ref, sc_ref, pd_ref, gd_ref, mk_ref, out_ref):
    pa = pa_ref[...]
    s = jnp.sum(pa, axis=0, keepdims=True)
    lane = lax.broadcasted_iota(jnp.int32, (1, 128), 1)
    w = jnp.zeros((1, 128), jnp.float32)
    for i in range(3):
        w = jnp.where(lane == i, _W_CHAMFER[i] / (_B * _NGT), w)
        w = jnp.where(lane == 3 + i,
                      _W_CHAMFER[i] * _W_CHAMFER_OPP / (_B * _NV[i]), w)
    total = jnp.sum(s * w)

    sc = jnp.sum(sc_ref[...], axis=0)
    r = lax.broadcasted_iota(jnp.int32, (_NTERMS, 16), 0)
    wvals = ([_W_EDGE / (_B * _NE[i]) for i in range(3)]
             + [_W_NORMAL / (_B * _NE[i]) for i in range(3)]
             + [_W_LAPLACE * _LAP_CONST[i] / (_B * _NV[i]) for i in range(3)]
             + [_W_MOVE * _LAP_CONST[i] / (_B * _NV[i]) for i in (1, 2)])
    wr = jnp.zeros((_NTERMS, 16), jnp.float32)
    for idx, wv in enumerate(wvals):
        wr = jnp.where(r == idx, wv, wr)
    total = total + jnp.sum(sc * wr)

    dd = pd_ref[...] - gd_ref[...]
    ax = jnp.abs(dd)
    hub = jnp.where(ax < 1.0, 0.5 * dd * dd, ax - 0.5)
    m = mk_ref[...] > 0.5
    sd = jnp.sum(jnp.where(m, hub, 0.0))
    cntm = jnp.sum(jnp.where(m, 1.0, 0.0))
    total = total + _W_DEPTH * sd / jnp.maximum(cntm, 1.0)
    out_ref[...] = jnp.zeros((1, 128), jnp.float32) + total


def _pack_coords(arrs, padval):
    cols = []
    for a, npad in zip(arrs, _NPAD):
        a = a.astype(jnp.float32)
        cols.append(jnp.pad(a, ((0, 0), (0, npad - a.shape[1]), (0, 0)),
                            constant_values=padval))
    cat = jnp.concatenate(cols, axis=1)
    cat = jnp.pad(cat, ((0, 0), (0, 0), (0, 5)))
    return jnp.transpose(cat, (0, 2, 1))


def _reg_call(predT, befT, gtnT, idx2, edges_all, lap_all):
    mesh = plsc.VectorSubcoreMesh(core_axis_name="c", subcore_axis_name="s")
    scratch = ([pltpu.VMEM((_MTOT,), jnp.float32)] * 6
               + [pltpu.VMEM((_NGTP,), jnp.float32)] * 3
               + [pltpu.VMEM((_MTOT,), jnp.int32)]
               + [pltpu.VMEM((_EVW,), jnp.int32)] * 2
               + [pltpu.VMEM((8 * _LVW,), jnp.int32)]
               + [pltpu.VMEM((16,), jnp.float32)] * _NTERMS
               + [pltpu.SemaphoreType.DMA])
    f = functools.partial(
        pl.kernel,
        mesh=mesh,
        out_type=jax.ShapeDtypeStruct((32 * _NTERMS * 16,), jnp.float32),
        scratch_types=scratch,
        compiler_params=pltpu.CompilerParams(needs_layout_passes=False),
    )(_reg_body)
    out = f(predT.reshape(-1), befT.reshape(-1), gtnT.reshape(-1),
            idx2.reshape(-1), edges_all[0], edges_all[1], lap_all.reshape(-1))
    return out.reshape(32, _NTERMS, 16)


def kernel(pred_coord_0, pred_coord_1, pred_coord_2,
           pred_coord_before_deform_0, pred_coord_before_deform_1,
           pred_coord_before_deform_2, pred_depth, gt_points, gt_normals,
           gt_images, gt_depth, mask, edges_0, edges_1, edges_2,
           laplace_idx_0, laplace_idx_1, laplace_idx_2):
    preds = (pred_coord_0, pred_coord_1, pred_coord_2)
    befs = (pred_coord_before_deform_0, pred_coord_before_deform_1,
            pred_coord_before_deform_2)
    edges = (edges_0, edges_1, edges_2)
    laps = (laplace_idx_0, laplace_idx_1, laplace_idx_2)

    predT = _pack_coords(preds, _PADV)
    befT = _pack_coords(befs, 0.0)

    gtp = jnp.pad(gt_points.astype(jnp.float32), ((0, 0), (0, 0), (0, 5)))
    gtp = jnp.pad(gtp, ((0, 0), (0, _NGTP - _NGT), (0, 0)),
                  constant_values=_PADV)
    gtnT = jnp.transpose(gt_normals.astype(jnp.float32), (0, 2, 1))
    gtnT = jnp.pad(gtnT, ((0, 0), (0, 0), (0, _NGTP - _NGT)))

    e_parts = []
    for e, ep in zip(edges, _EPAD):
        e = e.astype(jnp.int32)
        e_parts.append(jnp.pad(e, ((0, ep - e.shape[0]), (0, 0))))
    edges_all = jnp.transpose(jnp.concatenate(e_parts, axis=0), (1, 0))

    l_parts = []
    for lp, npad in zip(laps, _NPAD):
        lp8 = lp[:, :8].astype(jnp.int32)
        l_parts.append(jnp.pad(lp8, ((0, npad - lp8.shape[0]), (0, 0)),
                               constant_values=-1))
    lap_all = jnp.transpose(jnp.concatenate(l_parts, axis=0), (1, 0))

    pa, idx2 = pl.pallas_call(
        _chamfer_body,
        grid=(_B, _NT),
        in_specs=[
            pl.BlockSpec((1, _TN, 8), lambda b, t: (b, t, 0)),
            pl.BlockSpec((1, 8, _MTOT), lambda b, t: (b, 0, 0)),
        ],
        out_specs=[
            pl.BlockSpec((1, 8, 128), lambda b, t: (b, 0, 0)),
            pl.BlockSpec((1, 1, _MTOT), lambda b, t: (b, 0, 0)),
        ],
        out_shape=[
            jax.ShapeDtypeStruct((_B, 8, 128), jnp.float32),
            jax.ShapeDtypeStruct((_B, 1, _MTOT), jnp.int32),
        ],
        scratch_shapes=[pltpu.VMEM((1, _MTOT), jnp.float32)],
    )(gtp, predT)

    sc_out = _reg_call(predT, befT, gtnT, idx2, edges_all, lap_all)

    out = pl.pallas_call(
        _final_body,
        out_shape=jax.ShapeDtypeStruct((1, 128), jnp.float32),
    )(pa.reshape(_B * 8, 128), sc_out,
      pred_depth.reshape(_B, -1).astype(jnp.float32),
      gt_depth.reshape(_B, -1).astype(jnp.float32),
      mask.reshape(_B, -1).astype(jnp.float32))
    return out[0, 0]


# speedup vs baseline: 6.7240x; 1.1972x over previous
import functools

import jax
import jax.numpy as jnp
from jax import lax
from jax.experimental import pallas as pl
from jax.experimental.pallas import tpu as pltpu
from jax.experimental.pallas import tpu_sc as plsc

_W_CHAMFER = (1.0, 1.0, 1.0)
_W_CHAMFER_OPP = 0.55
_W_LAPLACE = 0.5
_W_MOVE = 0.1
_W_EDGE = 0.1
_W_NORMAL = 0.00016
_W_DEPTH = 1.0
_LAP_CONST = (0.2, 1.0, 1.0)

_B = 2
_NGT = 8000
_NGTP = 8192
_TN = 1024
_NT = _NGTP // _TN
_NV = (156, 618, 2466)
_NE = (462, 1848, 7392)
_NPAD = (256, 768, 2560)
_VOFF = (0, 256, 1024)
_MTOT = 3584
_EPAD = (512, 2048, 7424)
_EOFF = (0, 512, 2560)
_ETOT = 9984
_EW = tuple(e // 16 for e in _EPAD)
_ESEG = (0, 32, 160)
_EVW = 624
_NW = tuple(p // 16 for p in _NPAD)
_LSEG = (0, 16, 64)
_LVW = 224
_PADV = 1e6
_NTERMS = 11


def _chamfer_body(gt_ref, pred_ref, pa_ref, idx2_ref, min2_s):
    t = pl.program_id(1)
    g = gt_ref[0]
    p = pred_ref[0]
    d = lax.dot_general(g, p, (((1,), (0,)), ((), ())),
                        preferred_element_type=jnp.float32)

    rows = lax.broadcasted_iota(jnp.int32, (_TN, 1), 0) + t * _TN
    rvalid = rows < _NGT
    s1 = []
    for i in range(3):
        m1 = jnp.min(d[:, _VOFF[i]:_VOFF[i] + _NPAD[i]], axis=1, keepdims=True)
        s1.append(jnp.sum(jnp.where(rvalid, m1, 0.0)))

    tile_min = jnp.min(d, axis=0, keepdims=True)
    targ = jnp.argmin(d, axis=0).astype(jnp.int32).reshape(1, _MTOT) + t * _TN

    @pl.when(t == 0)
    def _():
        min2_s[...] = jnp.full((1, _MTOT), 1e30, jnp.float32)

    cur = min2_s[...]
    better = tile_min < cur
    min2_s[...] = jnp.where(better, tile_min, cur)
    idx2_ref[...] = jnp.where(better, targ,
                              idx2_ref[...].reshape(1, _MTOT)
                              ).reshape(1, 1, _MTOT)

    lane = lax.broadcasted_iota(jnp.int32, (1, 128), 1)
    vals = jnp.zeros((1, 128), jnp.float32)
    for i in range(3):
        vals = jnp.where(lane == i, s1[i], vals)

    @pl.when(t == 0)
    def _():
        pa_ref[...] = jnp.zeros((1, 8, 128), jnp.float32)

    @pl.when(t == _NT - 1)
    def _():
        m2 = jnp.where(better, tile_min, cur)
        v2 = jnp.zeros((1, 128), jnp.float32)
        for i in range(3):
            sl = m2[:, _VOFF[i]:_VOFF[i] + _NPAD[i]]
            li = lax.broadcasted_iota(jnp.int32, (1, _NPAD[i]), 1)
            s2 = jnp.sum(jnp.where(li < _NV[i], sl, 0.0))
            v2 = jnp.where(lane == 3 + i, s2, v2)
        subl2 = lax.broadcasted_iota(jnp.int32, (1, 8, 128), 1)
        pa_ref[...] = pa_ref[...] + jnp.where(subl2 == 1,
                                              v2.reshape(1, 1, 128), 0.0)

    subl = lax.broadcasted_iota(jnp.int32, (1, 8, 128), 1)
    v8 = jnp.where(subl == 0, vals.reshape(1, 1, 128), 0.0)
    pa_ref[...] = pa_ref[...] + v8


def _rsqrt16(x):
    i = lax.bitcast_convert_type(x, jnp.int32)
    i = jnp.int32(0x5F3759DF) - lax.shift_right_arithmetic(i, 1)
    y = lax.bitcast_convert_type(i, jnp.float32)
    for _ in range(3):
        y = y * (1.5 - 0.5 * x * y * y)
    return y


def _reg_body(predT, befT, gtnT, idx2, e0_hbm, e1_hbm, lap_hbm, out,
              px, py, pz, bx, by, bz, gnx, gny, gnz, i2v, e0_v, e1_v, lap_v,
              a0, a1, a2, a3, a4, a5, a6, a7, a8, a9, a10, sem):
    c = lax.axis_index("c")
    s = lax.axis_index("s")
    wid = s * 2 + c
    b = wid // 16
    w16 = wid % 16

    copies = []

    def cp(src, dst):
        copies.append(pltpu.async_copy(src, dst, sem))

    cp(predT.at[pl.ds((b * 8 + 0) * _MTOT, _MTOT)], px)
    cp(predT.at[pl.ds((b * 8 + 1) * _MTOT, _MTOT)], py)
    cp(predT.at[pl.ds((b * 8 + 2) * _MTOT, _MTOT)], pz)
    cp(befT.at[pl.ds((b * 8 + 0) * _MTOT, _MTOT)], bx)
    cp(befT.at[pl.ds((b * 8 + 1) * _MTOT, _MTOT)], by)
    cp(befT.at[pl.ds((b * 8 + 2) * _MTOT, _MTOT)], bz)
    cp(gtnT.at[pl.ds((b * 3 + 0) * _NGTP, _NGTP)], gnx)
    cp(gtnT.at[pl.ds((b * 3 + 1) * _NGTP, _NGTP)], gny)
    cp(gtnT.at[pl.ds((b * 3 + 2) * _NGTP, _NGTP)], gnz)
    cp(idx2.at[pl.ds(b * _MTOT, _MTOT)], i2v)
    for i in range(3):
        cp(e0_hbm.at[pl.ds(_EOFF[i] + w16 * _EW[i], _EW[i])],
           e0_v.at[pl.ds(_ESEG[i], _EW[i])])
        cp(e1_hbm.at[pl.ds(_EOFF[i] + w16 * _EW[i], _EW[i])],
           e1_v.at[pl.ds(_ESEG[i], _EW[i])])
        for k in range(8):
            cp(lap_hbm.at[pl.ds(k * _MTOT + _VOFF[i] + w16 * _NW[i], _NW[i])],
               lap_v.at[pl.ds(k * _LVW + _LSEG[i], _NW[i])])
    for cpy in copies:
        cpy.wait()

    zero = jnp.zeros((16,), jnp.float32)
    lane16 = lax.iota(jnp.int32, 16)
    acc_e = [zero, zero, zero]
    acc_n = [zero, zero, zero]
    acc_l = [zero, zero, zero]
    acc_m = [zero, zero, zero]

    for i in range(3):
        for j in range(_EW[i] // 16):
            gid = w16 * _EW[i] + j * 16 + lane16
            valid = gid < _NE[i]
            i0 = e0_v[pl.ds(_ESEG[i] + j * 16, 16)] + _VOFF[i]
            i1 = e1_v[pl.ds(_ESEG[i] + j * 16, 16)] + _VOFF[i]
            p0x = plsc.load_gather(px, [i0])
            p0y = plsc.load_gather(py, [i0])
            p0z = plsc.load_gather(pz, [i0])
            p1x = plsc.load_gather(px, [i1])
            p1y = plsc.load_gather(py, [i1])
            p1z = plsc.load_gather(pz, [i1])
            dx = p0x - p1x
            dy = p0y - p1y
            dz = p0z - p1z
            ss = dx * dx + dy * dy + dz * dz
            acc_e[i] = acc_e[i] + jnp.where(valid, ss, 0.0)
            nrm = ss * _rsqrt16(jnp.maximum(ss, 1e-30))
            inv = 1.0 / jnp.maximum(nrm, 1e-12)
            gi = plsc.load_gather(i2v, [i0])
            nxv = plsc.load_gather(gnx, [gi])
            nyv = plsc.load_gather(gny, [gi])
            nzv = plsc.load_gather(gnz, [gi])
            ssn = nxv * nxv + nyv * nyv + nzv * nzv
            nrmn = ssn * _rsqrt16(jnp.maximum(ssn, 1e-30))
            invn = 1.0 / jnp.maximum(nrmn, 1e-12)
            dot = (dx * nxv + dy * nyv + dz * nzv) * (inv * invn)
            acc_n[i] = acc_n[i] + jnp.where(valid, jnp.abs(dot), 0.0)

    for i in range(3):
        for j in range(_NW[i] // 16):
            vbase = w16 * _NW[i] + j * 16
            vid = vbase + lane16
            valid = vid < _NV[i]
            goff = _VOFF[i] + vbase
            dx = bx[pl.ds(goff, 16)] - px[pl.ds(goff, 16)]
            dy = by[pl.ds(goff, 16)] - py[pl.ds(goff, 16)]
            dz = bz[pl.ds(goff, 16)] - pz[pl.ds(goff, 16)]
            nsx = zero
            nsy = zero
            nsz = zero
            cnt = zero
            for k in range(8):
                nb = lap_v[pl.ds(k * _LVW + _LSEG[i] + j * 16, 16)]
                nvalid = nb >= 0
                safe = jnp.maximum(nb, 0) + _VOFF[i]
                gx = plsc.load_gather(bx, [safe]) - plsc.load_gather(px, [safe])
                gy = plsc.load_gather(by, [safe]) - plsc.load_gather(py, [safe])
                gz = plsc.load_gather(bz, [safe]) - plsc.load_gather(pz, [safe])
                nsx = nsx + jnp.where(nvalid, gx, 0.0)
                nsy = nsy + jnp.where(nvalid, gy, 0.0)
                nsz = nsz + jnp.where(nvalid, gz, 0.0)
                cnt = cnt + jnp.where(nvalid, 1.0, 0.0)
            cc = jnp.maximum(cnt, 1.0)
            ldx = dx - nsx / cc
            ldy = dy - nsy / cc
            ldz = dz - nsz / cc
            lsq = ldx * ldx + ldy * ldy + ldz * ldz
            acc_l[i] = acc_l[i] + jnp.where(valid, lsq, 0.0)
            if i > 0:
                mv = dx * dx + dy * dy + dz * dz
                acc_m[i] = acc_m[i] + jnp.where(valid, mv, 0.0)

    regs = [acc_e[0], acc_e[1], acc_e[2],
            acc_n[0], acc_n[1], acc_n[2],
            acc_l[0], acc_l[1], acc_l[2],
            acc_m[1], acc_m[2]]
    stores = [a0, a1, a2, a3, a4, a5, a6, a7, a8, a9, a10]
    out_copies = []
    for r in range(_NTERMS):
        stores[r][...] = regs[r]
        out_copies.append(pltpu.async_copy(
            stores[r], out.at[pl.ds((wid * _NTERMS + r) * 16, 16)], sem))
    for cpy in out_copies:
        cpy.wait()


def _final_body(pa_ref, sc_ref, pd_ref, gd_ref, mk_ref, out_ref):
    pa = pa_ref[...]
    s = jnp.sum(pa, axis=0, keepdims=True)
    lane = lax.broadcasted_iota(jnp.int32, (1, 128), 1)
    w = jnp.zeros((1, 128), jnp.float32)
    for i in range(3):
        w = jnp.where(lane == i, _W_CHAMFER[i] / (_B * _NGT), w)
        w = jnp.where(lane == 3 + i,
                      _W_CHAMFER[i] * _W_CHAMFER_OPP / (_B * _NV[i]), w)
    total = jnp.sum(s * w)

    sc = jnp.sum(sc_ref[...], axis=0)
    r = lax.broadcasted_iota(jnp.int32, (_NTERMS, 16), 0)
    wvals = ([_W_EDGE / (_B * _NE[i]) for i in range(3)]
             + [_W_NORMAL / (_B * _NE[i]) for i in range(3)]
             + [_W_LAPLACE * _LAP_CONST[i] / (_B * _NV[i]) for i in range(3)]
             + [_W_MOVE * _LAP_CONST[i] / (_B * _NV[i]) for i in (1, 2)])
    wr = jnp.zeros((_NTERMS, 16), jnp.float32)
    for idx, wv in enumerate(wvals):
        wr = jnp.where(r == idx, wv, wr)
    total = total + jnp.sum(sc * wr)

    dd = pd_ref[...] - gd_ref[...]
    ax = jnp.abs(dd)
    hub = jnp.where(ax < 1.0, 0.5 * dd * dd, ax - 0.5)
    m = mk_ref[...] > 0.5
    sd = jnp.sum(jnp.where(m, hub, 0.0))
    cntm = jnp.sum(jnp.where(m, 1.0, 0.0))
    total = total + _W_DEPTH * sd / jnp.maximum(cntm, 1.0)
    out_ref[...] = jnp.zeros((1, 128), jnp.float32) + total


def _pack_coords(arrs, padval):
    cols = []
    for a, npad in zip(arrs, _NPAD):
        a = a.astype(jnp.float32)
        cols.append(jnp.pad(a, ((0, 0), (0, npad - a.shape[1]), (0, 0)),
                            constant_values=padval))
    cat = jnp.concatenate(cols, axis=1)
    cat = jnp.pad(cat, ((0, 0), (0, 0), (0, 5)))
    return jnp.transpose(cat, (0, 2, 1))


def _reg_call(predT, befT, gtnT, idx2, edges_all, lap_all):
    mesh = plsc.VectorSubcoreMesh(core_axis_name="c", subcore_axis_name="s")
    scratch = ([pltpu.VMEM((_MTOT,), jnp.float32)] * 6
               + [pltpu.VMEM((_NGTP,), jnp.float32)] * 3
               + [pltpu.VMEM((_MTOT,), jnp.int32)]
               + [pltpu.VMEM((_EVW,), jnp.int32)] * 2
               + [pltpu.VMEM((8 * _LVW,), jnp.int32)]
               + [pltpu.VMEM((16,), jnp.float32)] * _NTERMS
               + [pltpu.SemaphoreType.DMA])
    f = functools.partial(
        pl.kernel,
        mesh=mesh,
        out_type=jax.ShapeDtypeStruct((32 * _NTERMS * 16,), jnp.float32),
        scratch_types=scratch,
        compiler_params=pltpu.CompilerParams(needs_layout_passes=False),
    )(_reg_body)
    out = f(predT.reshape(-1), befT.reshape(-1), gtnT.reshape(-1),
            idx2.reshape(-1), edges_all[0], edges_all[1], lap_all.reshape(-1))
    return out.reshape(32, _NTERMS, 16)


def kernel(pred_coord_0, pred_coord_1, pred_coord_2,
           pred_coord_before_deform_0, pred_coord_before_deform_1,
           pred_coord_before_deform_2, pred_depth, gt_points, gt_normals,
           gt_images, gt_depth, mask, edges_0, edges_1, edges_2,
           laplace_idx_0, laplace_idx_1, laplace_idx_2):
    preds = (pred_coord_0, pred_coord_1, pred_coord_2)
    befs = (pred_coord_before_deform_0, pred_coord_before_deform_1,
            pred_coord_before_deform_2)
    edges = (edges_0, edges_1, edges_2)
    laps = (laplace_idx_0, laplace_idx_1, laplace_idx_2)

    predT = _pack_coords(preds, _PADV)
    befT = _pack_coords(befs, 0.0)

    pc = jnp.transpose(predT[:, :3, :], (0, 2, 1))
    p2c = jnp.sum(pc * pc, axis=-1, keepdims=True)
    onesp = jnp.ones_like(p2c)
    pred_ext = jnp.concatenate(
        [-2.0 * pc, onesp, p2c, jnp.zeros((_B, _MTOT, 11), jnp.float32)],
        axis=-1)
    pred_ext = jnp.transpose(pred_ext, (0, 2, 1))

    gtc = jnp.pad(gt_points.astype(jnp.float32),
                  ((0, 0), (0, _NGTP - _NGT), (0, 0)),
                  constant_values=_PADV)
    g2c = jnp.sum(gtc * gtc, axis=-1, keepdims=True)
    gt_ext = jnp.concatenate(
        [gtc, g2c, jnp.ones_like(g2c),
         jnp.zeros((_B, _NGTP, 11), jnp.float32)], axis=-1)
    gtnT = jnp.transpose(gt_normals.astype(jnp.float32), (0, 2, 1))
    gtnT = jnp.pad(gtnT, ((0, 0), (0, 0), (0, _NGTP - _NGT)))

    e_parts = []
    for e, ep in zip(edges, _EPAD):
        e = e.astype(jnp.int32)
        e_parts.append(jnp.pad(e, ((0, ep - e.shape[0]), (0, 0))))
    edges_all = jnp.transpose(jnp.concatenate(e_parts, axis=0), (1, 0))

    l_parts = []
    for lp, npad in zip(laps, _NPAD):
        lp8 = lp[:, :8].astype(jnp.int32)
        l_parts.append(jnp.pad(lp8, ((0, npad - lp8.shape[0]), (0, 0)),
                               constant_values=-1))
    lap_all = jnp.transpose(jnp.concatenate(l_parts, axis=0), (1, 0))

    pa, idx2 = pl.pallas_call(
        _chamfer_body,
        grid=(_B, _NT),
        in_specs=[
            pl.BlockSpec((1, _TN, 16), lambda b, t: (b, t, 0)),
            pl.BlockSpec((1, 16, _MTOT), lambda b, t: (b, 0, 0)),
        ],
        out_specs=[
            pl.BlockSpec((1, 8, 128), lambda b, t: (b, 0, 0)),
            pl.BlockSpec((1, 1, _MTOT), lambda b, t: (b, 0, 0)),
        ],
        out_shape=[
            jax.ShapeDtypeStruct((_B, 8, 128), jnp.float32),
            jax.ShapeDtypeStruct((_B, 1, _MTOT), jnp.int32),
        ],
        scratch_shapes=[pltpu.VMEM((1, _MTOT), jnp.float32)],
    )(gt_ext, pred_ext)

    sc_out = _reg_call(predT, befT, gtnT, idx2, edges_all, lap_all)

    out = pl.pallas_call(
        _final_body,
        out_shape=jax.ShapeDtypeStruct((1, 128), jnp.float32),
    )(pa.reshape(_B * 8, 128), sc_out,
      pred_depth.reshape(_B, -1).astype(jnp.float32),
      gt_depth.reshape(_B, -1).astype(jnp.float32),
      mask.reshape(_B, -1).astype(jnp.float32))
    return out[0, 0]


# speedup vs baseline: 7.1645x; 1.0655x over previous
import functools

import jax
import jax.numpy as jnp
from jax import lax
from jax.experimental import pallas as pl
from jax.experimental.pallas import tpu as pltpu
from jax.experimental.pallas import tpu_sc as plsc

_W_CHAMFER = (1.0, 1.0, 1.0)
_W_CHAMFER_OPP = 0.55
_W_LAPLACE = 0.5
_W_MOVE = 0.1
_W_EDGE = 0.1
_W_NORMAL = 0.00016
_W_DEPTH = 1.0
_LAP_CONST = (0.2, 1.0, 1.0)

_B = 2
_NGT = 8000
_TN = 1000
_NT = _NGT // _TN
_NV = (156, 618, 2466)
_NE = (462, 1848, 7392)
_NPAD = (256, 768, 2560)
_VOFF = (0, 256, 1024)
_MTOT = 3584
_EPAD = (512, 2048, 7424)
_EOFF = (0, 512, 2560)
_ETOT = 9984
_EW = tuple(e // 16 for e in _EPAD)
_ESEG = (0, 32, 160)
_EVW = 624
_NW = tuple(p // 16 for p in _NPAD)
_LSEG = (0, 16, 64)
_LVW = 224
_PADV = 1e6
_NTERMS = 11
_SOFF = (0, 156, 774, 3240, 3396, 4014, 6480)
_ROW = 6480 + 8000


def _chamfer_body(gt_ref, pred_ref, pa_ref, idx2_ref, min2_s):
    t = pl.program_id(1)
    g = gt_ref[0]
    p = pred_ref[0]
    g2 = jnp.sum(g * g, axis=1, keepdims=True)
    ge = jnp.concatenate([g, g2, jnp.ones_like(g2)], axis=1)
    d = lax.dot_general(ge, p, (((1,), (1,)), ((), ())),
                        preferred_element_type=jnp.float32)

    s1 = []
    for i in range(3):
        m1 = jnp.min(d[:, _VOFF[i]:_VOFF[i] + _NPAD[i]], axis=1, keepdims=True)
        s1.append(jnp.sum(m1))

    rowk = lax.broadcasted_iota(jnp.int32, (_TN, _MTOT), 0) + (t * _TN)
    key = (lax.bitcast_convert_type(d, jnp.int32) & jnp.int32(-8192)) | rowk
    tile_key = jnp.min(key, axis=0, keepdims=True)

    @pl.when(t == 0)
    def _():
        min2_s[...] = jnp.full((1, _MTOT), jnp.int32(0x7F000000))

    newk = jnp.minimum(min2_s[...], tile_key)
    min2_s[...] = newk

    lane = lax.broadcasted_iota(jnp.int32, (1, 128), 1)
    vals = jnp.zeros((1, 128), jnp.float32)
    for i in range(3):
        vals = jnp.where(lane == i, s1[i], vals)

    @pl.when(t == 0)
    def _():
        pa_ref[...] = jnp.zeros((1, 8, 128), jnp.float32)

    @pl.when(t == _NT - 1)
    def _():
        idx2_ref[...] = (newk & 8191).reshape(1, 1, _MTOT)
        m2 = lax.bitcast_convert_type(newk & jnp.int32(-8192), jnp.float32)
        v2 = jnp.zeros((1, 128), jnp.float32)
        for i in range(3):
            sl = m2[:, _VOFF[i]:_VOFF[i] + _NPAD[i]]
            li = lax.broadcasted_iota(jnp.int32, (1, _NPAD[i]), 1)
            s2 = jnp.sum(jnp.where(li < _NV[i], sl, 0.0))
            v2 = jnp.where(lane == 3 + i, s2, v2)
        subl2 = lax.broadcasted_iota(jnp.int32, (1, 8, 128), 1)
        pa_ref[...] = pa_ref[...] + jnp.where(subl2 == 1,
                                              v2.reshape(1, 1, 128), 0.0)

    subl = lax.broadcasted_iota(jnp.int32, (1, 8, 128), 1)
    v8 = jnp.where(subl == 0, vals.reshape(1, 1, 128), 0.0)
    pa_ref[...] = pa_ref[...] + v8


def _rsqrt16(x):
    i = lax.bitcast_convert_type(x, jnp.int32)
    i = jnp.int32(0x5F3759DF) - lax.shift_right_arithmetic(i, 1)
    y = lax.bitcast_convert_type(i, jnp.float32)
    for _ in range(3):
        y = y * (1.5 - 0.5 * x * y * y)
    return y


def _reg_body(allc, idx2, e0_hbm, e1_hbm, lap_hbm, out,
              allv, i2v, e0_v, e1_v, lap_v,
              a0, a1, a2, a3, a4, a5, a6, a7, a8, a9, a10, sem):
    c = lax.axis_index("c")
    s = lax.axis_index("s")
    wid = s * 2 + c
    b = wid // 16
    w16 = wid % 16

    copies = []

    def cp(src, dst):
        copies.append(pltpu.async_copy(src, dst, sem))

    cp(allc.at[pl.ds(b * (_ROW * 3), _ROW * 3)], allv)
    cp(idx2.at[pl.ds(b * _MTOT, _MTOT)], i2v)
    for i in range(3):
        cp(e0_hbm.at[pl.ds(_EOFF[i] + w16 * _EW[i], _EW[i])],
           e0_v.at[pl.ds(_ESEG[i], _EW[i])])
        cp(e1_hbm.at[pl.ds(_EOFF[i] + w16 * _EW[i], _EW[i])],
           e1_v.at[pl.ds(_ESEG[i], _EW[i])])
        for k in range(8):
            cp(lap_hbm.at[pl.ds(k * _MTOT + _VOFF[i] + w16 * _NW[i], _NW[i])],
               lap_v.at[pl.ds(k * _LVW + _LSEG[i], _NW[i])])
    for cpy in copies:
        cpy.wait()

    zero = jnp.zeros((16,), jnp.float32)
    lane16 = lax.iota(jnp.int32, 16)
    acc_e = [zero, zero, zero]
    acc_n = [zero, zero, zero]
    acc_l = [zero, zero, zero]
    acc_m = [zero, zero, zero]

    def g3(vidx, base):
        a = vidx * 3 + base
        return (plsc.load_gather(allv, [a]),
                plsc.load_gather(allv, [a + 1]),
                plsc.load_gather(allv, [a + 2]))

    for i in range(3):
        pbase = _SOFF[i] * 3
        for j in range(_EW[i] // 16):
            gid = w16 * _EW[i] + j * 16 + lane16
            valid = gid < _NE[i]
            i0 = e0_v[pl.ds(_ESEG[i] + j * 16, 16)]
            i1 = e1_v[pl.ds(_ESEG[i] + j * 16, 16)]
            p0x, p0y, p0z = g3(i0, pbase)
            p1x, p1y, p1z = g3(i1, pbase)
            dx = p0x - p1x
            dy = p0y - p1y
            dz = p0z - p1z
            ss = dx * dx + dy * dy + dz * dz
            acc_e[i] = acc_e[i] + jnp.where(valid, ss, 0.0)
            nrm = ss * _rsqrt16(jnp.maximum(ss, 1e-30))
            inv = 1.0 / jnp.maximum(nrm, 1e-12)
            gi = plsc.load_gather(i2v, [i0 + _VOFF[i]])
            nxv, nyv, nzv = g3(gi, _SOFF[6] * 3)
            ssn = nxv * nxv + nyv * nyv + nzv * nzv
            nrmn = ssn * _rsqrt16(jnp.maximum(ssn, 1e-30))
            invn = 1.0 / jnp.maximum(nrmn, 1e-12)
            dot = (dx * nxv + dy * nyv + dz * nzv) * (inv * invn)
            acc_n[i] = acc_n[i] + jnp.where(valid, jnp.abs(dot), 0.0)

    for i in range(3):
        pbase = _SOFF[i] * 3
        bbase = _SOFF[3 + i] * 3
        for j in range(_NW[i] // 16):
            vbase = w16 * _NW[i] + j * 16
            vid = jnp.minimum(vbase + lane16, _NV[i] - 1)
            valid = (vbase + lane16) < _NV[i]
            cx, cy, cz = g3(vid, pbase)
            ex, ey, ez = g3(vid, bbase)
            dx = ex - cx
            dy = ey - cy
            dz = ez - cz
            nsx = zero
            nsy = zero
            nsz = zero
            cnt = zero
            for k in range(8):
                nb = lap_v[pl.ds(k * _LVW + _LSEG[i] + j * 16, 16)]
                nvalid = nb >= 0
                safe = jnp.maximum(nb, 0)
                gpx, gpy, gpz = g3(safe, pbase)
                gbx, gby, gbz = g3(safe, bbase)
                nsx = nsx + jnp.where(nvalid, gbx - gpx, 0.0)
                nsy = nsy + jnp.where(nvalid, gby - gpy, 0.0)
                nsz = nsz + jnp.where(nvalid, gbz - gpz, 0.0)
                cnt = cnt + jnp.where(nvalid, 1.0, 0.0)
            cc = jnp.maximum(cnt, 1.0)
            ldx = dx - nsx / cc
            ldy = dy - nsy / cc
            ldz = dz - nsz / cc
            lsq = ldx * ldx + ldy * ldy + ldz * ldz
            acc_l[i] = acc_l[i] + jnp.where(valid, lsq, 0.0)
            if i > 0:
                mv = dx * dx + dy * dy + dz * dz
                acc_m[i] = acc_m[i] + jnp.where(valid, mv, 0.0)

    regs = [acc_e[0], acc_e[1], acc_e[2],
            acc_n[0], acc_n[1], acc_n[2],
            acc_l[0], acc_l[1], acc_l[2],
            acc_m[1], acc_m[2]]
    stores = [a0, a1, a2, a3, a4, a5, a6, a7, a8, a9, a10]
    out_copies = []
    for r in range(_NTERMS):
        stores[r][...] = regs[r]
        out_copies.append(pltpu.async_copy(
            stores[r], out.at[pl.ds((wid * _NTERMS + r) * 16, 16)], sem))
    for cpy in out_copies:
        cpy.wait()


def _final_body(pa_ref, sc_ref, pd_ref, gd_ref, mk_ref, out_ref):
    pa = pa_ref[...]
    s = jnp.sum(pa, axis=0, keepdims=True)
    lane = lax.broadcasted_iota(jnp.int32, (1, 128), 1)
    w = jnp.zeros((1, 128), jnp.float32)
    for i in range(3):
        w = jnp.where(lane == i, _W_CHAMFER[i] / (_B * _NGT), w)
        w = jnp.where(lane == 3 + i,
                      _W_CHAMFER[i] * _W_CHAMFER_OPP / (_B * _NV[i]), w)
    total = jnp.sum(s * w)

    sc = jnp.sum(sc_ref[...], axis=0)
    r = lax.broadcasted_iota(jnp.int32, (_NTERMS, 16), 0)
    wvals = ([_W_EDGE / (_B * _NE[i]) for i in range(3)]
             + [_W_NORMAL / (_B * _NE[i]) for i in range(3)]
             + [_W_LAPLACE * _LAP_CONST[i] / (_B * _NV[i]) for i in range(3)]
             + [_W_MOVE * _LAP_CONST[i] / (_B * _NV[i]) for i in (1, 2)])
    wr = jnp.zeros((_NTERMS, 16), jnp.float32)
    for idx, wv in enumerate(wvals):
        wr = jnp.where(r == idx, wv, wr)
    total = total + jnp.sum(sc * wr)

    dd = pd_ref[...] - gd_ref[...]
    ax = jnp.abs(dd)
    hub = jnp.where(ax < 1.0, 0.5 * dd * dd, ax - 0.5)
    m = mk_ref[...] > 0.5
    sd = jnp.sum(jnp.where(m, hub, 0.0))
    cntm = jnp.sum(jnp.where(m, 1.0, 0.0))
    total = total + _W_DEPTH * sd / jnp.maximum(cntm, 1.0)
    out_ref[...] = jnp.zeros((1, 128), jnp.float32) + total


def _reg_call(allc, idx2, e0, e1, lap_flat):
    mesh = plsc.VectorSubcoreMesh(core_axis_name="c", subcore_axis_name="s")
    scratch = ([pltpu.VMEM((_ROW * 3,), jnp.float32)]
               + [pltpu.VMEM((_MTOT,), jnp.int32)]
               + [pltpu.VMEM((_EVW,), jnp.int32)] * 2
               + [pltpu.VMEM((8 * _LVW,), jnp.int32)]
               + [pltpu.VMEM((16,), jnp.float32)] * _NTERMS
               + [pltpu.SemaphoreType.DMA])
    f = functools.partial(
        pl.kernel,
        mesh=mesh,
        out_type=jax.ShapeDtypeStruct((32 * _NTERMS * 16,), jnp.float32),
        scratch_types=scratch,
        compiler_params=pltpu.CompilerParams(needs_layout_passes=False),
    )(_reg_body)
    out = f(allc, idx2.reshape(-1), e0, e1, lap_flat)
    return out.reshape(32, _NTERMS, 16)


def kernel(pred_coord_0, pred_coord_1, pred_coord_2,
           pred_coord_before_deform_0, pred_coord_before_deform_1,
           pred_coord_before_deform_2, pred_depth, gt_points, gt_normals,
           gt_images, gt_depth, mask, edges_0, edges_1, edges_2,
           laplace_idx_0, laplace_idx_1, laplace_idx_2):
    preds = (pred_coord_0, pred_coord_1, pred_coord_2)
    befs = (pred_coord_before_deform_0, pred_coord_before_deform_1,
            pred_coord_before_deform_2)
    edges = (edges_0, edges_1, edges_2)
    laps = (laplace_idx_0, laplace_idx_1, laplace_idx_2)

    pcat = jnp.concatenate(
        [jnp.pad(a.astype(jnp.float32),
                 ((0, 0), (0, npad - a.shape[1]), (0, 0)),
                 constant_values=_PADV)
         for a, npad in zip(preds, _NPAD)], axis=1)
    p2c = jnp.sum(pcat * pcat, axis=-1, keepdims=True)
    pred_ext = jnp.concatenate(
        [-2.0 * pcat, jnp.ones_like(p2c), p2c], axis=-1)

    e0 = jnp.concatenate(
        [jnp.pad(e.astype(jnp.int32)[:, 0], (0, ep - e.shape[0]))
         for e, ep in zip(edges, _EPAD)])
    e1 = jnp.concatenate(
        [jnp.pad(e.astype(jnp.int32)[:, 1], (0, ep - e.shape[0]))
         for e, ep in zip(edges, _EPAD)])
    lap_flat = jnp.concatenate(
        [jnp.pad(lp[:, k].astype(jnp.int32),
                 (0, npad - lp.shape[0]), constant_values=-1)
         for k in range(8)
         for lp, npad in zip(laps, _NPAD)])

    pa, idx2 = pl.pallas_call(
        _chamfer_body,
        grid=(_B, _NT),
        in_specs=[
            pl.BlockSpec((1, _TN, 3), lambda b, t: (b, t, 0)),
            pl.BlockSpec((1, _MTOT, 5), lambda b, t: (b, 0, 0)),
        ],
        out_specs=[
            pl.BlockSpec((1, 8, 128), lambda b, t: (b, 0, 0)),
            pl.BlockSpec((1, 1, _MTOT), lambda b, t: (b, 0, 0)),
        ],
        out_shape=[
            jax.ShapeDtypeStruct((_B, 8, 128), jnp.float32),
            jax.ShapeDtypeStruct((_B, 1, _MTOT), jnp.int32),
        ],
        scratch_shapes=[pltpu.VMEM((1, _MTOT), jnp.int32)],
    )(gt_points, pred_ext)

    allc = jnp.concatenate(
        [p.astype(jnp.float32) for p in preds]
        + [bf.astype(jnp.float32) for bf in befs]
        + [gt_normals.astype(jnp.float32)], axis=1).reshape(-1)
    sc_out = _reg_call(allc, idx2, e0, e1, lap_flat)

    out = pl.pallas_call(
        _final_body,
        out_shape=jax.ShapeDtypeStruct((1, 128), jnp.float32),
    )(pa.reshape(_B * 8, 128), sc_out,
      pred_depth.reshape(_B, -1).astype(jnp.float32),
      gt_depth.reshape(_B, -1).astype(jnp.float32),
      mask.reshape(_B, -1).astype(jnp.float32))
    return out[0, 0]


# speedup vs baseline: 7.2370x; 1.0101x over previous
import functools

import jax
import jax.numpy as jnp
from jax import lax
from jax.experimental import pallas as pl
from jax.experimental.pallas import tpu as pltpu
from jax.experimental.pallas import tpu_sc as plsc

_W_CHAMFER = (1.0, 1.0, 1.0)
_W_CHAMFER_OPP = 0.55
_W_LAPLACE = 0.5
_W_MOVE = 0.1
_W_EDGE = 0.1
_W_NORMAL = 0.00016
_W_DEPTH = 1.0
_LAP_CONST = (0.2, 1.0, 1.0)

_B = 2
_NGT = 8000
_TN = 1000
_NT = _NGT // _TN
_NV = (156, 618, 2466)
_NE = (462, 1848, 7392)
_NPAD = (256, 768, 2560)
_VOFF = (0, 256, 1024)
_MTOT = 3584
_EPAD = (512, 2048, 7424)
_EOFF = (0, 512, 2560)
_ETOT = 9984
_EW = tuple(e // 16 for e in _EPAD)
_ESEG = (0, 32, 160)
_EVW = 624
_NW = tuple(p // 16 for p in _NPAD)
_LSEG = (0, 16, 64)
_LVW = 224
_PADV = 1e6
_NTERMS = 11
_SOFF = (0, 156, 774, 3240, 3396, 4014, 6480)
_ROW = 6480 + 8000


def _chamfer_body(gt_ref, pred_ref, pa_ref, idx2_ref, min2_s):
    t = pl.program_id(1)
    g = gt_ref[0]
    p = pred_ref[0]
    g2 = jnp.sum(g * g, axis=1, keepdims=True)
    ge = jnp.concatenate([g, g2, jnp.ones_like(g2)], axis=1)
    d = lax.dot_general(ge, p, (((1,), (1,)), ((), ())),
                        preferred_element_type=jnp.float32)

    s1 = []
    for i in range(3):
        m1 = jnp.min(d[:, _VOFF[i]:_VOFF[i] + _NPAD[i]], axis=1, keepdims=True)
        s1.append(jnp.sum(m1))

    rowk = lax.broadcasted_iota(jnp.int32, (_TN, _MTOT), 0) + (t * _TN)
    key = (lax.bitcast_convert_type(d, jnp.int32) & jnp.int32(-8192)) | rowk
    tile_key = jnp.min(key, axis=0, keepdims=True)

    @pl.when(t == 0)
    def _():
        min2_s[...] = jnp.full((1, _MTOT), jnp.int32(0x7F000000))

    newk = jnp.minimum(min2_s[...], tile_key)
    min2_s[...] = newk

    lane = lax.broadcasted_iota(jnp.int32, (1, 128), 1)
    vals = jnp.zeros((1, 128), jnp.float32)
    for i in range(3):
        vals = jnp.where(lane == i, s1[i], vals)

    @pl.when(t == 0)
    def _():
        pa_ref[...] = jnp.zeros((1, 8, 128), jnp.float32)

    @pl.when(t == _NT - 1)
    def _():
        idx2_ref[...] = (newk & 8191).reshape(1, 1, _MTOT)
        m2 = lax.bitcast_convert_type(newk & jnp.int32(-8192), jnp.float32)
        v2 = jnp.zeros((1, 128), jnp.float32)
        for i in range(3):
            sl = m2[:, _VOFF[i]:_VOFF[i] + _NPAD[i]]
            li = lax.broadcasted_iota(jnp.int32, (1, _NPAD[i]), 1)
            s2 = jnp.sum(jnp.where(li < _NV[i], sl, 0.0))
            v2 = jnp.where(lane == 3 + i, s2, v2)
        subl2 = lax.broadcasted_iota(jnp.int32, (1, 8, 128), 1)
        pa_ref[...] = pa_ref[...] + jnp.where(subl2 == 1,
                                              v2.reshape(1, 1, 128), 0.0)

    subl = lax.broadcasted_iota(jnp.int32, (1, 8, 128), 1)
    v8 = jnp.where(subl == 0, vals.reshape(1, 1, 128), 0.0)
    pa_ref[...] = pa_ref[...] + v8


def _rsqrt16(x):
    i = lax.bitcast_convert_type(x, jnp.int32)
    i = jnp.int32(0x5F3759DF) - lax.shift_right_arithmetic(i, 1)
    y = lax.bitcast_convert_type(i, jnp.float32)
    for _ in range(3):
        y = y * (1.5 - 0.5 * x * y * y)
    return y


def _reg_a_body(allc, e0_hbm, e1_hbm, lap_hbm, out,
                allv, e0_v, e1_v, lap_v,
                a0, a1, a2, a3, a4, a5, a6, a7, sem):
    c = lax.axis_index("c")
    s = lax.axis_index("s")
    wid = s * 2 + c
    b = wid // 16
    w16 = wid % 16

    copies = []

    def cp(src, dst):
        copies.append(pltpu.async_copy(src, dst, sem))

    cp(allc.at[pl.ds(b * (_ROW * 3), _SOFF[6] * 3)], allv)
    for i in range(3):
        cp(e0_hbm.at[pl.ds(_EOFF[i] + w16 * _EW[i], _EW[i])],
           e0_v.at[pl.ds(_ESEG[i], _EW[i])])
        cp(e1_hbm.at[pl.ds(_EOFF[i] + w16 * _EW[i], _EW[i])],
           e1_v.at[pl.ds(_ESEG[i], _EW[i])])
        for k in range(8):
            cp(lap_hbm.at[pl.ds(k * _MTOT + _VOFF[i] + w16 * _NW[i], _NW[i])],
               lap_v.at[pl.ds(k * _LVW + _LSEG[i], _NW[i])])
    for cpy in copies:
        cpy.wait()

    zero = jnp.zeros((16,), jnp.float32)
    lane16 = lax.iota(jnp.int32, 16)
    acc_e = [zero, zero, zero]
    acc_l = [zero, zero, zero]
    acc_m = [zero, zero, zero]

    def g3(vidx, base):
        a = vidx * 3 + base
        return (plsc.load_gather(allv, [a]),
                plsc.load_gather(allv, [a + 1]),
                plsc.load_gather(allv, [a + 2]))

    for i in range(3):
        pbase = _SOFF[i] * 3
        for j in range(_EW[i] // 16):
            gid = w16 * _EW[i] + j * 16 + lane16
            valid = gid < _NE[i]
            i0 = e0_v[pl.ds(_ESEG[i] + j * 16, 16)]
            i1 = e1_v[pl.ds(_ESEG[i] + j * 16, 16)]
            p0x, p0y, p0z = g3(i0, pbase)
            p1x, p1y, p1z = g3(i1, pbase)
            dx = p0x - p1x
            dy = p0y - p1y
            dz = p0z - p1z
            ss = dx * dx + dy * dy + dz * dz
            acc_e[i] = acc_e[i] + jnp.where(valid, ss, 0.0)

    for i in range(3):
        pbase = _SOFF[i] * 3
        bbase = _SOFF[3 + i] * 3
        for j in range(_NW[i] // 16):
            vbase = w16 * _NW[i] + j * 16
            vid = jnp.minimum(vbase + lane16, _NV[i] - 1)
            valid = (vbase + lane16) < _NV[i]
            cx, cy, cz = g3(vid, pbase)
            ex, ey, ez = g3(vid, bbase)
            dx = ex - cx
            dy = ey - cy
            dz = ez - cz
            nsx = zero
            nsy = zero
            nsz = zero
            cnt = zero
            for k in range(8):
                nb = lap_v[pl.ds(k * _LVW + _LSEG[i] + j * 16, 16)]
                nvalid = nb >= 0
                safe = jnp.maximum(nb, 0)
                gpx, gpy, gpz = g3(safe, pbase)
                gbx, gby, gbz = g3(safe, bbase)
                nsx = nsx + jnp.where(nvalid, gbx - gpx, 0.0)
                nsy = nsy + jnp.where(nvalid, gby - gpy, 0.0)
                nsz = nsz + jnp.where(nvalid, gbz - gpz, 0.0)
                cnt = cnt + jnp.where(nvalid, 1.0, 0.0)
            cc = jnp.maximum(cnt, 1.0)
            ldx = dx - nsx / cc
            ldy = dy - nsy / cc
            ldz = dz - nsz / cc
            lsq = ldx * ldx + ldy * ldy + ldz * ldz
            acc_l[i] = acc_l[i] + jnp.where(valid, lsq, 0.0)
            if i > 0:
                mv = dx * dx + dy * dy + dz * dz
                acc_m[i] = acc_m[i] + jnp.where(valid, mv, 0.0)

    regs = [acc_e[0], acc_e[1], acc_e[2],
            acc_l[0], acc_l[1], acc_l[2],
            acc_m[1], acc_m[2]]
    stores = [a0, a1, a2, a3, a4, a5, a6, a7]
    out_copies = []
    for r in range(8):
        stores[r][...] = regs[r]
        out_copies.append(pltpu.async_copy(
            stores[r], out.at[pl.ds((wid * 8 + r) * 16, 16)], sem))
    for cpy in out_copies:
        cpy.wait()


def _reg_b_body(allc, idx2, e0_hbm, e1_hbm, out,
                allv, i2v, e0_v, e1_v, a0, a1, a2, sem):
    c = lax.axis_index("c")
    s = lax.axis_index("s")
    wid = s * 2 + c
    b = wid // 16
    w16 = wid % 16

    copies = []

    def cp(src, dst):
        copies.append(pltpu.async_copy(src, dst, sem))

    cp(allc.at[pl.ds(b * (_ROW * 3), _ROW * 3)], allv)
    cp(idx2.at[pl.ds(b * _MTOT, _MTOT)], i2v)
    for i in range(3):
        cp(e0_hbm.at[pl.ds(_EOFF[i] + w16 * _EW[i], _EW[i])],
           e0_v.at[pl.ds(_ESEG[i], _EW[i])])
        cp(e1_hbm.at[pl.ds(_EOFF[i] + w16 * _EW[i], _EW[i])],
           e1_v.at[pl.ds(_ESEG[i], _EW[i])])
    for cpy in copies:
        cpy.wait()

    zero = jnp.zeros((16,), jnp.float32)
    lane16 = lax.iota(jnp.int32, 16)
    acc_n = [zero, zero, zero]

    def g3(vidx, base):
        a = vidx * 3 + base
        return (plsc.load_gather(allv, [a]),
                plsc.load_gather(allv, [a + 1]),
                plsc.load_gather(allv, [a + 2]))

    for i in range(3):
        pbase = _SOFF[i] * 3
        for j in range(_EW[i] // 16):
            gid = w16 * _EW[i] + j * 16 + lane16
            valid = gid < _NE[i]
            i0 = e0_v[pl.ds(_ESEG[i] + j * 16, 16)]
            i1 = e1_v[pl.ds(_ESEG[i] + j * 16, 16)]
            p0x, p0y, p0z = g3(i0, pbase)
            p1x, p1y, p1z = g3(i1, pbase)
            dx = p0x - p1x
            dy = p0y - p1y
            dz = p0z - p1z
            ss = dx * dx + dy * dy + dz * dz
            nrm = ss * _rsqrt16(jnp.maximum(ss, 1e-30))
            inv = 1.0 / jnp.maximum(nrm, 1e-12)
            gi = plsc.load_gather(i2v, [i0 + _VOFF[i]])
            nxv, nyv, nzv = g3(gi, _SOFF[6] * 3)
            ssn = nxv * nxv + nyv * nyv + nzv * nzv
            nrmn = ssn * _rsqrt16(jnp.maximum(ssn, 1e-30))
            invn = 1.0 / jnp.maximum(nrmn, 1e-12)
            dot = (dx * nxv + dy * nyv + dz * nzv) * (inv * invn)
            acc_n[i] = acc_n[i] + jnp.where(valid, jnp.abs(dot), 0.0)

    stores = [a0, a1, a2]
    out_copies = []
    for r in range(3):
        stores[r][...] = acc_n[r]
        out_copies.append(pltpu.async_copy(
            stores[r], out.at[pl.ds((wid * 3 + r) * 16, 16)], sem))
    for cpy in out_copies:
        cpy.wait()


def _final_body(pa_ref, sa_ref, sb_ref, pd_ref, gd_ref, mk_ref, out_ref):
    pa = pa_ref[...]
    s = jnp.sum(pa, axis=(0, 1)).reshape(1, 128)
    lane = lax.broadcasted_iota(jnp.int32, (1, 128), 1)
    w = jnp.zeros((1, 128), jnp.float32)
    for i in range(3):
        w = jnp.where(lane == i, _W_CHAMFER[i] / (_B * _NGT), w)
        w = jnp.where(lane == 3 + i,
                      _W_CHAMFER[i] * _W_CHAMFER_OPP / (_B * _NV[i]), w)
    total = jnp.sum(s * w)

    sa = jnp.sum(sa_ref[...], axis=0)
    ra = lax.broadcasted_iota(jnp.int32, (8, 16), 0)
    wavals = ([_W_EDGE / (_B * _NE[i]) for i in range(3)]
              + [_W_LAPLACE * _LAP_CONST[i] / (_B * _NV[i]) for i in range(3)]
              + [_W_MOVE * _LAP_CONST[i] / (_B * _NV[i]) for i in (1, 2)])
    wa = jnp.zeros((8, 16), jnp.float32)
    for idx, wv in enumerate(wavals):
        wa = jnp.where(ra == idx, wv, wa)
    total = total + jnp.sum(sa * wa)

    sb = jnp.sum(sb_ref[...], axis=0)
    rb = lax.broadcasted_iota(jnp.int32, (3, 16), 0)
    wb = jnp.zeros((3, 16), jnp.float32)
    for i in range(3):
        wb = jnp.where(rb == i, _W_NORMAL / (_B * _NE[i]), wb)
    total = total + jnp.sum(sb * wb)

    dd = pd_ref[...] - gd_ref[...]
    ax = jnp.abs(dd)
    hub = jnp.where(ax < 1.0, 0.5 * dd * dd, ax - 0.5)
    m = mk_ref[...] > 0.5
    sd = jnp.sum(jnp.where(m, hub, 0.0))
    cntm = jnp.sum(jnp.where(m, 1.0, 0.0))
    total = total + _W_DEPTH * sd / jnp.maximum(cntm, 1.0)
    out_ref[...] = jnp.zeros((1, 128), jnp.float32) + total


def _reg_call_a(allc, e0, e1, lap_flat):
    mesh = plsc.VectorSubcoreMesh(core_axis_name="c", subcore_axis_name="s")
    scratch = ([pltpu.VMEM((_SOFF[6] * 3,), jnp.float32)]
               + [pltpu.VMEM((_EVW,), jnp.int32)] * 2
               + [pltpu.VMEM((8 * _LVW,), jnp.int32)]
               + [pltpu.VMEM((16,), jnp.float32)] * 8
               + [pltpu.SemaphoreType.DMA])
    f = functools.partial(
        pl.kernel,
        mesh=mesh,
        out_type=jax.ShapeDtypeStruct((32 * 8 * 16,), jnp.float32),
        scratch_types=scratch,
        compiler_params=pltpu.CompilerParams(needs_layout_passes=False),
    )(_reg_a_body)
    return f(allc, e0, e1, lap_flat).reshape(32, 8, 16)


def _reg_call_b(allc, idx2, e0, e1):
    mesh = plsc.VectorSubcoreMesh(core_axis_name="c", subcore_axis_name="s")
    scratch = ([pltpu.VMEM((_ROW * 3,), jnp.float32)]
               + [pltpu.VMEM((_MTOT,), jnp.int32)]
               + [pltpu.VMEM((_EVW,), jnp.int32)] * 2
               + [pltpu.VMEM((16,), jnp.float32)] * 3
               + [pltpu.SemaphoreType.DMA])
    f = functools.partial(
        pl.kernel,
        mesh=mesh,
        out_type=jax.ShapeDtypeStruct((32 * 3 * 16,), jnp.float32),
        scratch_types=scratch,
        compiler_params=pltpu.CompilerParams(needs_layout_passes=False),
    )(_reg_b_body)
    return f(allc, idx2.reshape(-1), e0, e1).reshape(32, 3, 16)


def kernel(pred_coord_0, pred_coord_1, pred_coord_2,
           pred_coord_before_deform_0, pred_coord_before_deform_1,
           pred_coord_before_deform_2, pred_depth, gt_points, gt_normals,
           gt_images, gt_depth, mask, edges_0, edges_1, edges_2,
           laplace_idx_0, laplace_idx_1, laplace_idx_2):
    preds = (pred_coord_0, pred_coord_1, pred_coord_2)
    befs = (pred_coord_before_deform_0, pred_coord_before_deform_1,
            pred_coord_before_deform_2)
    edges = (edges_0, edges_1, edges_2)
    laps = (laplace_idx_0, laplace_idx_1, laplace_idx_2)

    pcat = jnp.concatenate(
        [jnp.pad(a.astype(jnp.float32),
                 ((0, 0), (0, npad - a.shape[1]), (0, 0)),
                 constant_values=_PADV)
         for a, npad in zip(preds, _NPAD)], axis=1)
    p2c = jnp.sum(pcat * pcat, axis=-1, keepdims=True)
    pred_ext = jnp.concatenate(
        [-2.0 * pcat, jnp.ones_like(p2c), p2c], axis=-1)

    e0 = jnp.concatenate(
        [jnp.pad(e.astype(jnp.int32)[:, 0], (0, ep - e.shape[0]))
         for e, ep in zip(edges, _EPAD)])
    e1 = jnp.concatenate(
        [jnp.pad(e.astype(jnp.int32)[:, 1], (0, ep - e.shape[0]))
         for e, ep in zip(edges, _EPAD)])
    lap_flat = jnp.concatenate(
        [jnp.pad(lp[:, k].astype(jnp.int32),
                 (0, npad - lp.shape[0]), constant_values=-1)
         for k in range(8)
         for lp, npad in zip(laps, _NPAD)])

    allc = jnp.concatenate(
        [p.astype(jnp.float32) for p in preds]
        + [bf.astype(jnp.float32) for bf in befs]
        + [gt_normals.astype(jnp.float32)], axis=1).reshape(-1)
    sc_a = _reg_call_a(allc, e0, e1, lap_flat)

    pa, idx2 = pl.pallas_call(
        _chamfer_body,
        grid=(_B, _NT),
        in_specs=[
            pl.BlockSpec((1, _TN, 3), lambda b, t: (b, t, 0)),
            pl.BlockSpec((1, _MTOT, 5), lambda b, t: (b, 0, 0)),
        ],
        out_specs=[
            pl.BlockSpec((1, 8, 128), lambda b, t: (b, 0, 0)),
            pl.BlockSpec((1, 1, _MTOT), lambda b, t: (b, 0, 0)),
        ],
        out_shape=[
            jax.ShapeDtypeStruct((_B, 8, 128), jnp.float32),
            jax.ShapeDtypeStruct((_B, 1, _MTOT), jnp.int32),
        ],
        scratch_shapes=[pltpu.VMEM((1, _MTOT), jnp.int32)],
    )(gt_points, pred_ext)

    sc_b = _reg_call_b(allc, idx2, e0, e1)

    out = pl.pallas_call(
        _final_body,
        out_shape=jax.ShapeDtypeStruct((1, 128), jnp.float32),
    )(pa, sc_a, sc_b,
      pred_depth.reshape(_B, -1).astype(jnp.float32),
      gt_depth.reshape(_B, -1).astype(jnp.float32),
      mask.reshape(_B, -1).astype(jnp.float32))
    return out[0, 0]


# speedup vs baseline: 7.3527x; 1.0160x over previous
import functools

import jax
import jax.numpy as jnp
from jax import lax
from jax.experimental import pallas as pl
from jax.experimental.pallas import tpu as pltpu
from jax.experimental.pallas import tpu_sc as plsc

_W_CHAMFER = (1.0, 1.0, 1.0)
_W_CHAMFER_OPP = 0.55
_W_LAPLACE = 0.5
_W_MOVE = 0.1
_W_EDGE = 0.1
_W_NORMAL = 0.00016
_W_DEPTH = 1.0
_LAP_CONST = (0.2, 1.0, 1.0)

_B = 2
_NGT = 8000
_TN = 1000
_NT = _NGT // _TN
_NV = (156, 618, 2466)
_NE = (462, 1848, 7392)
_NPAD = (256, 768, 2560)
_VOFF = (0, 256, 1024)
_MTOT = 3584
_EPAD = (512, 2048, 7424)
_EOFF = (0, 512, 2560)
_ETOT = 9984
_EW = tuple(e // 16 for e in _EPAD)
_ESEG = (0, 32, 160)
_EVW = 624
_NW = tuple(p // 16 for p in _NPAD)
_LSEG = (0, 16, 64)
_LVW = 224
_PADV = 1e6
_NTERMS = 11
_SOFF = (0, 156, 774, 3240, 3396, 4014, 6480)
_ROW = 6480 + 8000


def _chamfer_body(gt_ref, pred_ref, pa_ref, idx2_ref, min2_s):
    t = pl.program_id(1)
    g = gt_ref[0]
    p = pred_ref[0]
    g2 = jnp.sum(g * g, axis=1, keepdims=True)
    ge = jnp.concatenate([g, g2, jnp.ones_like(g2)], axis=1)
    d = lax.dot_general(ge, p, (((1,), (1,)), ((), ())),
                        preferred_element_type=jnp.float32)

    s1 = []
    for i in range(3):
        m1 = jnp.min(d[:, _VOFF[i]:_VOFF[i] + _NPAD[i]], axis=1, keepdims=True)
        s1.append(jnp.sum(m1))

    rowk = lax.broadcasted_iota(jnp.int32, (_TN, _MTOT), 0) + (t * _TN)
    key = (lax.bitcast_convert_type(d, jnp.int32) & jnp.int32(-8192)) | rowk
    tile_key = jnp.min(key, axis=0, keepdims=True)

    @pl.when(t == 0)
    def _():
        min2_s[...] = jnp.full((1, _MTOT), jnp.int32(0x7F000000))

    newk = jnp.minimum(min2_s[...], tile_key)
    min2_s[...] = newk

    lane = lax.broadcasted_iota(jnp.int32, (1, 128), 1)
    vals = jnp.zeros((1, 128), jnp.float32)
    for i in range(3):
        vals = jnp.where(lane == i, s1[i], vals)

    @pl.when(t == 0)
    def _():
        pa_ref[...] = jnp.zeros((1, 8, 128), jnp.float32)

    @pl.when(t == _NT - 1)
    def _():
        idx2_ref[pl.ds(0, _MTOT)] = (newk & 8191).reshape(_MTOT)
        m2 = lax.bitcast_convert_type(newk & jnp.int32(-8192), jnp.float32)
        v2 = jnp.zeros((1, 128), jnp.float32)
        for i in range(3):
            sl = m2[:, _VOFF[i]:_VOFF[i] + _NPAD[i]]
            li = lax.broadcasted_iota(jnp.int32, (1, _NPAD[i]), 1)
            s2 = jnp.sum(jnp.where(li < _NV[i], sl, 0.0))
            v2 = jnp.where(lane == 3 + i, s2, v2)
        subl2 = lax.broadcasted_iota(jnp.int32, (1, 8, 128), 1)
        pa_ref[...] = pa_ref[...] + jnp.where(subl2 == 1,
                                              v2.reshape(1, 1, 128), 0.0)

    subl = lax.broadcasted_iota(jnp.int32, (1, 8, 128), 1)
    v8 = jnp.where(subl == 0, vals.reshape(1, 1, 128), 0.0)
    pa_ref[...] = pa_ref[...] + v8


def _rsqrt16(x):
    i = lax.bitcast_convert_type(x, jnp.int32)
    i = jnp.int32(0x5F3759DF) - lax.shift_right_arithmetic(i, 1)
    y = lax.bitcast_convert_type(i, jnp.float32)
    for _ in range(3):
        y = y * (1.5 - 0.5 * x * y * y)
    return y


def _reg_a_body(allc, itab, out,
                allv, e0_v, e1_v, lap_v,
                a0, a1, a2, a3, a4, a5, a6, a7, sem):
    c = lax.axis_index("c")
    s = lax.axis_index("s")
    wid = s * 2 + c
    b = wid // 16
    w16 = wid % 16

    copies = []

    def cp(src, dst):
        copies.append(pltpu.async_copy(src, dst, sem))

    cp(allc.at[pl.ds(b * (_ROW * 3), _SOFF[6] * 3)], allv)
    for i in range(3):
        cp(itab.at[pl.ds(_EOFF[i] + w16 * _EW[i], _EW[i])],
           e0_v.at[pl.ds(_ESEG[i], _EW[i])])
        cp(itab.at[pl.ds(_ETOT + _EOFF[i] + w16 * _EW[i], _EW[i])],
           e1_v.at[pl.ds(_ESEG[i], _EW[i])])
        for k in range(8):
            cp(itab.at[pl.ds(2 * _ETOT + k * _MTOT + _VOFF[i]
                             + w16 * _NW[i], _NW[i])],
               lap_v.at[pl.ds(k * _LVW + _LSEG[i], _NW[i])])
    for cpy in copies:
        cpy.wait()

    zero = jnp.zeros((16,), jnp.float32)
    lane16 = lax.iota(jnp.int32, 16)
    acc_e = [zero, zero, zero]
    acc_l = [zero, zero, zero]
    acc_m = [zero, zero, zero]

    def g3(vidx, base):
        a = vidx * 3 + base
        return (plsc.load_gather(allv, [a]),
                plsc.load_gather(allv, [a + 1]),
                plsc.load_gather(allv, [a + 2]))

    for i in range(3):
        pbase = _SOFF[i] * 3
        for j in range(_EW[i] // 16):
            gid = w16 * _EW[i] + j * 16 + lane16
            valid = gid < _NE[i]
            i0 = e0_v[pl.ds(_ESEG[i] + j * 16, 16)]
            i1 = e1_v[pl.ds(_ESEG[i] + j * 16, 16)]
            p0x, p0y, p0z = g3(i0, pbase)
            p1x, p1y, p1z = g3(i1, pbase)
            dx = p0x - p1x
            dy = p0y - p1y
            dz = p0z - p1z
            ss = dx * dx + dy * dy + dz * dz
            acc_e[i] = acc_e[i] + jnp.where(valid, ss, 0.0)

    for i in range(3):
        pbase = _SOFF[i] * 3
        bbase = _SOFF[3 + i] * 3
        for j in range(_NW[i] // 16):
            vbase = w16 * _NW[i] + j * 16
            vid = jnp.minimum(vbase + lane16, _NV[i] - 1)
            valid = (vbase + lane16) < _NV[i]
            cx, cy, cz = g3(vid, pbase)
            ex, ey, ez = g3(vid, bbase)
            dx = ex - cx
            dy = ey - cy
            dz = ez - cz
            nsx = zero
            nsy = zero
            nsz = zero
            cnt = zero
            for k in range(8):
                nb = lap_v[pl.ds(k * _LVW + _LSEG[i] + j * 16, 16)]
                nvalid = nb >= 0
                safe = jnp.maximum(nb, 0)
                gpx, gpy, gpz = g3(safe, pbase)
                gbx, gby, gbz = g3(safe, bbase)
                nsx = nsx + jnp.where(nvalid, gbx - gpx, 0.0)
                nsy = nsy + jnp.where(nvalid, gby - gpy, 0.0)
                nsz = nsz + jnp.where(nvalid, gbz - gpz, 0.0)
                cnt = cnt + jnp.where(nvalid, 1.0, 0.0)
            cc = jnp.maximum(cnt, 1.0)
            ldx = dx - nsx / cc
            ldy = dy - nsy / cc
            ldz = dz - nsz / cc
            lsq = ldx * ldx + ldy * ldy + ldz * ldz
            acc_l[i] = acc_l[i] + jnp.where(valid, lsq, 0.0)
            if i > 0:
                mv = dx * dx + dy * dy + dz * dz
                acc_m[i] = acc_m[i] + jnp.where(valid, mv, 0.0)

    regs = [acc_e[0], acc_e[1], acc_e[2],
            acc_l[0], acc_l[1], acc_l[2],
            acc_m[1], acc_m[2]]
    stores = [a0, a1, a2, a3, a4, a5, a6, a7]
    out_copies = []
    for r in range(8):
        stores[r][...] = regs[r]
        out_copies.append(pltpu.async_copy(
            stores[r], out.at[pl.ds((wid * 8 + r) * 16, 16)], sem))
    for cpy in out_copies:
        cpy.wait()


def _reg_b_body(allc, idx2, itab, out,
                allv, i2v, e0_v, e1_v, a0, a1, a2, a3, sem):
    c = lax.axis_index("c")
    s = lax.axis_index("s")
    wid = s * 2 + c
    b = wid // 16
    w16 = wid % 16

    copies = []

    def cp(src, dst):
        copies.append(pltpu.async_copy(src, dst, sem))

    cp(allc.at[pl.ds(b * (_ROW * 3), _ROW * 3)], allv)
    cp(idx2.at[pl.ds(b * 4096, _MTOT)], i2v)
    for i in range(3):
        cp(itab.at[pl.ds(_EOFF[i] + w16 * _EW[i], _EW[i])],
           e0_v.at[pl.ds(_ESEG[i], _EW[i])])
        cp(itab.at[pl.ds(_ETOT + _EOFF[i] + w16 * _EW[i], _EW[i])],
           e1_v.at[pl.ds(_ESEG[i], _EW[i])])
    for cpy in copies:
        cpy.wait()

    zero = jnp.zeros((16,), jnp.float32)
    lane16 = lax.iota(jnp.int32, 16)
    acc_n = [zero, zero, zero]

    def g3(vidx, base):
        a = vidx * 3 + base
        return (plsc.load_gather(allv, [a]),
                plsc.load_gather(allv, [a + 1]),
                plsc.load_gather(allv, [a + 2]))

    for i in range(3):
        pbase = _SOFF[i] * 3
        for j in range(_EW[i] // 16):
            gid = w16 * _EW[i] + j * 16 + lane16
            valid = gid < _NE[i]
            i0 = e0_v[pl.ds(_ESEG[i] + j * 16, 16)]
            i1 = e1_v[pl.ds(_ESEG[i] + j * 16, 16)]
            p0x, p0y, p0z = g3(i0, pbase)
            p1x, p1y, p1z = g3(i1, pbase)
            dx = p0x - p1x
            dy = p0y - p1y
            dz = p0z - p1z
            ss = dx * dx + dy * dy + dz * dz
            nrm = ss * _rsqrt16(jnp.maximum(ss, 1e-30))
            inv = 1.0 / jnp.maximum(nrm, 1e-12)
            gi = plsc.load_gather(i2v, [i0 + _VOFF[i]])
            nxv, nyv, nzv = g3(gi, _SOFF[6] * 3)
            ssn = nxv * nxv + nyv * nyv + nzv * nzv
            nrmn = ssn * _rsqrt16(jnp.maximum(ssn, 1e-30))
            invn = 1.0 / jnp.maximum(nrmn, 1e-12)
            dot = (dx * nxv + dy * nyv + dz * nzv) * (inv * invn)
            acc_n[i] = acc_n[i] + jnp.where(valid, jnp.abs(dot), 0.0)

    stores = [a0, a1, a2, a3]
    vals = acc_n + [zero]
    out_copies = []
    for r in range(4):
        stores[r][...] = vals[r]
        out_copies.append(pltpu.async_copy(
            stores[r], out.at[pl.ds((wid * 4 + r) * 16, 16)], sem))
    for cpy in out_copies:
        cpy.wait()


def _final_body(pa_ref, sa_ref, sb_ref, pd_ref, gd_ref, mk_ref, out_ref):
    pa = pa_ref[...]
    s = jnp.sum(pa, axis=(0, 1)).reshape(1, 128)
    lane = lax.broadcasted_iota(jnp.int32, (1, 128), 1)
    w = jnp.zeros((1, 128), jnp.float32)
    for i in range(3):
        w = jnp.where(lane == i, _W_CHAMFER[i] / (_B * _NGT), w)
        w = jnp.where(lane == 3 + i,
                      _W_CHAMFER[i] * _W_CHAMFER_OPP / (_B * _NV[i]), w)
    total = jnp.sum(s * w)

    sa = sa_ref[...]
    ra = (lax.shift_right_logical(
        lax.broadcasted_iota(jnp.int32, (32 * 8 * 16,), 0), 4)) & 7
    wavals = ([_W_EDGE / (_B * _NE[i]) for i in range(3)]
              + [_W_LAPLACE * _LAP_CONST[i] / (_B * _NV[i]) for i in range(3)]
              + [_W_MOVE * _LAP_CONST[i] / (_B * _NV[i]) for i in (1, 2)])
    wa = jnp.zeros((32 * 8 * 16,), jnp.float32)
    for idx, wv in enumerate(wavals):
        wa = jnp.where(ra == idx, wv, wa)
    total = total + jnp.sum(sa * wa)

    sb = sb_ref[...]
    rb = (lax.shift_right_logical(
        lax.broadcasted_iota(jnp.int32, (32 * 4 * 16,), 0), 4)) & 3
    wb = jnp.zeros((32 * 4 * 16,), jnp.float32)
    for i in range(3):
        wb = jnp.where(rb == i, _W_NORMAL / (_B * _NE[i]), wb)
    total = total + jnp.sum(sb * wb)

    dd = pd_ref[...] - gd_ref[...]
    ax = jnp.abs(dd)
    hub = jnp.where(ax < 1.0, 0.5 * dd * dd, ax - 0.5)
    m = mk_ref[...] > 0.5
    sd = jnp.sum(jnp.where(m, hub, 0.0))
    cntm = jnp.sum(jnp.where(m, 1.0, 0.0))
    total = total + _W_DEPTH * sd / jnp.maximum(cntm, 1.0)
    out_ref[...] = jnp.zeros((1, 128), jnp.float32) + total


def _reg_call_a(allc, itab):
    mesh = plsc.VectorSubcoreMesh(core_axis_name="c", subcore_axis_name="s")
    scratch = ([pltpu.VMEM((_SOFF[6] * 3,), jnp.float32)]
               + [pltpu.VMEM((_EVW,), jnp.int32)] * 2
               + [pltpu.VMEM((8 * _LVW,), jnp.int32)]
               + [pltpu.VMEM((16,), jnp.float32)] * 8
               + [pltpu.SemaphoreType.DMA])
    f = functools.partial(
        pl.kernel,
        mesh=mesh,
        out_type=jax.ShapeDtypeStruct((32 * 8 * 16,), jnp.float32),
        scratch_types=scratch,
        compiler_params=pltpu.CompilerParams(needs_layout_passes=False),
    )(_reg_a_body)
    return f(allc, itab)


def _reg_call_b(allc, idx2, itab):
    mesh = plsc.VectorSubcoreMesh(core_axis_name="c", subcore_axis_name="s")
    scratch = ([pltpu.VMEM((_ROW * 3,), jnp.float32)]
               + [pltpu.VMEM((_MTOT,), jnp.int32)]
               + [pltpu.VMEM((_EVW,), jnp.int32)] * 2
               + [pltpu.VMEM((16,), jnp.float32)] * 4
               + [pltpu.SemaphoreType.DMA])
    f = functools.partial(
        pl.kernel,
        mesh=mesh,
        out_type=jax.ShapeDtypeStruct((32 * 4 * 16,), jnp.float32),
        scratch_types=scratch,
        compiler_params=pltpu.CompilerParams(needs_layout_passes=False),
    )(_reg_b_body)
    return f(allc, idx2, itab)


def kernel(pred_coord_0, pred_coord_1, pred_coord_2,
           pred_coord_before_deform_0, pred_coord_before_deform_1,
           pred_coord_before_deform_2, pred_depth, gt_points, gt_normals,
           gt_images, gt_depth, mask, edges_0, edges_1, edges_2,
           laplace_idx_0, laplace_idx_1, laplace_idx_2):
    preds = (pred_coord_0, pred_coord_1, pred_coord_2)
    befs = (pred_coord_before_deform_0, pred_coord_before_deform_1,
            pred_coord_before_deform_2)
    edges = (edges_0, edges_1, edges_2)
    laps = (laplace_idx_0, laplace_idx_1, laplace_idx_2)

    pcat = jnp.concatenate(
        [jnp.pad(a.astype(jnp.float32),
                 ((0, 0), (0, npad - a.shape[1]), (0, 0)),
                 constant_values=_PADV)
         for a, npad in zip(preds, _NPAD)], axis=1)
    p2c = jnp.sum(pcat * pcat, axis=-1, keepdims=True)
    pred_ext = jnp.concatenate(
        [-2.0 * pcat, jnp.ones_like(p2c), p2c], axis=-1)

    itab = jnp.concatenate(
        [jnp.pad(e.astype(jnp.int32)[:, 0], (0, ep - e.shape[0]))
         for e, ep in zip(edges, _EPAD)]
        + [jnp.pad(e.astype(jnp.int32)[:, 1], (0, ep - e.shape[0]))
           for e, ep in zip(edges, _EPAD)]
        + [jnp.pad(lp[:, k].astype(jnp.int32),
                   (0, npad - lp.shape[0]), constant_values=-1)
           for k in range(8)
           for lp, npad in zip(laps, _NPAD)])

    allc = jnp.concatenate(
        [p.astype(jnp.float32) for p in preds]
        + [bf.astype(jnp.float32) for bf in befs]
        + [gt_normals.astype(jnp.float32)], axis=1).reshape(-1)
    sc_a = _reg_call_a(allc, itab)

    pa, idx2 = pl.pallas_call(
        _chamfer_body,
        grid=(_B, _NT),
        in_specs=[
            pl.BlockSpec((1, _TN, 3), lambda b, t: (b, t, 0)),
            pl.BlockSpec((1, _MTOT, 5), lambda b, t: (b, 0, 0)),
        ],
        out_specs=[
            pl.BlockSpec((1, 8, 128), lambda b, t: (b, 0, 0)),
            pl.BlockSpec((4096,), lambda b, t: (b,)),
        ],
        out_shape=[
            jax.ShapeDtypeStruct((_B, 8, 128), jnp.float32),
            jax.ShapeDtypeStruct((_B * 4096,), jnp.int32),
        ],
        scratch_shapes=[pltpu.VMEM((1, _MTOT), jnp.int32)],
    )(gt_points, pred_ext)

    sc_b = _reg_call_b(allc, idx2, itab)

    out = pl.pallas_call(
        _final_body,
        out_shape=jax.ShapeDtypeStruct((1, 128), jnp.float32),
    )(pa, sc_a, sc_b,
      pred_depth.reshape(_B, -1).astype(jnp.float32),
      gt_depth.reshape(_B, -1).astype(jnp.float32),
      mask.reshape(_B, -1).astype(jnp.float32))
    return out[0, 0]
